# baseline (speedup 1.0000x reference)
import functools

import jax
import jax.numpy as jnp
from jax import lax
from jax.experimental import pallas as pl
from jax.experimental.pallas import tpu as pltpu
from jax.experimental.pallas import tpu_sc as plsc

N = 10000
E = 320000
D = 128
H = 128

NC = 2
NS = 16
NW = NC * NS
EW = E // NW
NR = 640
SK = 80
NB = EW // SK
ROWS_PER_TILE = N // NS

_f32 = jnp.float32
_i32 = jnp.int32


_BLK = 1000
_GRID = N // _BLK


def _dense1_body(x_ref, w_ref, av_ref, we_ref, h_ref, al_ref, ce_ref):
    h = jnp.dot(x_ref[...], w_ref[...], preferred_element_type=_f32)
    h_ref[...] = h
    al_ref[...] = jnp.dot(h, av_ref[...], preferred_element_type=_f32)
    ce_ref[...] = jnp.dot(we_ref[...], av_ref[...], preferred_element_type=_f32)


def _dense1(x, w, av, we):
    return pl.pallas_call(
        _dense1_body,
        grid=(_GRID,),
        in_specs=[
            pl.BlockSpec((_BLK, D), lambda i: (i, 0)),
            pl.BlockSpec((D, H), lambda i: (0, 0)),
            pl.BlockSpec((H, 8), lambda i: (0, 0)),
            pl.BlockSpec((1, H), lambda i: (0, 0)),
        ],
        out_specs=[
            pl.BlockSpec((_BLK, H), lambda i: (i, 0)),
            pl.BlockSpec((_BLK, 8), lambda i: (i, 0)),
            pl.BlockSpec((1, 8), lambda i: (0, 0)),
        ],
        out_shape=[
            jax.ShapeDtypeStruct((N, H), _f32),
            jax.ShapeDtypeStruct((N, 8), _f32),
            jax.ShapeDtypeStruct((1, 8), _f32),
        ],
    )(x, w, av, we)


def _dense2_body(p0_ref, p1_ref, b_ref, w_ref, av_ref, we_ref,
                 h_ref, al_ref, ce_ref):
    xin = jnp.maximum(p0_ref[...] + p1_ref[...] + b_ref[...], 0.0)
    h = jnp.dot(xin, w_ref[...], preferred_element_type=_f32)
    h_ref[...] = h
    al_ref[...] = jnp.dot(h, av_ref[...], preferred_element_type=_f32)
    ce_ref[...] = jnp.dot(we_ref[...], av_ref[...], preferred_element_type=_f32)


def _dense2(p0, p1, b, w, av, we):
    return pl.pallas_call(
        _dense2_body,
        grid=(_GRID,),
        in_specs=[
            pl.BlockSpec((_BLK, H), lambda i: (i, 0)),
            pl.BlockSpec((_BLK, H), lambda i: (i, 0)),
            pl.BlockSpec((1, H), lambda i: (0, 0)),
            pl.BlockSpec((H, H), lambda i: (0, 0)),
            pl.BlockSpec((H, 8), lambda i: (0, 0)),
            pl.BlockSpec((1, H), lambda i: (0, 0)),
        ],
        out_specs=[
            pl.BlockSpec((_BLK, H), lambda i: (i, 0)),
            pl.BlockSpec((_BLK, 8), lambda i: (i, 0)),
            pl.BlockSpec((1, 8), lambda i: (0, 0)),
        ],
        out_shape=[
            jax.ShapeDtypeStruct((N, H), _f32),
            jax.ShapeDtypeStruct((N, 8), _f32),
            jax.ShapeDtypeStruct((1, 8), _f32),
        ],
    )(p0, p1, b, w, av, we)


def _final_body(p0_ref, p1_ref, b_ref, out_ref):
    out_ref[...] = p0_ref[...] + p1_ref[...] + b_ref[...]


def _final(p0, p1, b):
    return pl.pallas_call(
        _final_body,
        grid=(_GRID,),
        in_specs=[
            pl.BlockSpec((_BLK, H), lambda i: (i, 0)),
            pl.BlockSpec((_BLK, H), lambda i: (i, 0)),
            pl.BlockSpec((1, H), lambda i: (0, 0)),
        ],
        out_specs=pl.BlockSpec((_BLK, H), lambda i: (i, 0)),
        out_shape=jax.ShapeDtypeStruct((N, H), _f32),
    )(p0, p1, b)


_MESH = plsc.VectorSubcoreMesh(core_axis_name="c", subcore_axis_name="s")


@functools.partial(
    pl.kernel,
    mesh=_MESH,
    compiler_params=pltpu.CompilerParams(
        needs_layout_passes=False, use_tc_tiling_on_sc=False,
        disable_bounds_checks=True),
    out_type=[
        jax.ShapeDtypeStruct((E,), _f32),
        jax.ShapeDtypeStruct((NC, NR, 16), _f32),
    ],
    scratch_types=[
        pltpu.VMEM((EW,), _i32),
        pltpu.VMEM((EW,), _i32),
        pltpu.VMEM((EW,), _f32),
        pltpu.VMEM((N // 16, 16), _f32),
        pltpu.VMEM((N // 16, 16), _f32),
        pltpu.VMEM((NR, 16), _f32),
        pltpu.VMEM((16,), _f32),
        pltpu.VMEM((5, 128), _i32),
        pltpu.VMEM_SHARED((NR, 16), _f32),
        pltpu.SemaphoreType.DMA,
    ],
)
def _edge_alpha(src_hbm, dst_hbm, ew_hbm, asrc_hbm, adst_hbm, ce_hbm,
                ex_hbm, den_hbm,
                src_v, dst_v, ew_v, as_v, ad_v, den_v, ce_v, idx_v,
                den_sh, dsem):
    c_ax = lax.axis_index("c")
    s_ax = lax.axis_index("s")
    wid = s_ax * NC + c_ax
    base = wid * EW

    _ins = [(src_hbm.at[pl.ds(base, EW)], src_v),
            (dst_hbm.at[pl.ds(base, EW)], dst_v),
            (ew_hbm.at[pl.ds(base, EW)], ew_v),
            (asrc_hbm, as_v), (adst_hbm, ad_v), (ce_hbm, ce_v)]
    for _s, _d in _ins:
        pltpu.async_copy(_s, _d, dsem)
    for _s, _d in _ins:
        pltpu.make_async_copy(_s, _d, dsem).wait()
    c_e = ce_v[...]

    for j in range(5):
        for t in range(8):
            idx_v[j, pl.ds(t * 16, 16)] = (
                lax.broadcasted_iota(_i32, (16,), 0) + (j * 128 + t * 16)
            )

    def _zero(i, carry):
        den_v[i, :] = jnp.zeros((16,), _f32)
        return carry

    lax.fori_loop(0, NR, _zero, 0, unroll=8)

    @pl.when(s_ax == 0)
    def _init_shared():
        pltpu.sync_copy(den_v, den_sh)

    plsc.subcore_barrier()

    def _edges(i, carry):
        sl = pl.ds(i * 16, 16)
        si = src_v[sl]
        di = dst_v[sl]
        si_r = lax.shift_right_logical(si, 4)
        si_c = lax.bitwise_and(si, 15)
        di_r = lax.shift_right_logical(di, 4)
        di_c = lax.bitwise_and(di, 15)
        a = (plsc.load_gather(as_v, [si_r, si_c])
             + plsc.load_gather(ad_v, [di_r, di_c])
             + c_e * ew_v[sl])
        a = jnp.where(a > 0.0, a, 0.2 * a)
        exv = jnp.exp(a)
        ew_v[sl] = exv
        plsc.addupdate_scatter(den_v, [di_r, di_c], exv)
        return carry

    lax.fori_loop(0, EW // 16, _edges, 0, unroll=5)

    pltpu.sync_copy(ew_v, ex_hbm.at[pl.ds(base, EW)])

    for j in range(5):
        pltpu.sync_copy(den_v.at[pl.ds(j * 128, 128)],
                        den_sh.at[idx_v.at[j]], add=True)

    plsc.subcore_barrier()

    @pl.when(s_ax == 0)
    def _write_out():
        pltpu.sync_copy(den_sh, den_hbm.at[c_ax])


@functools.partial(
    pl.kernel,
    mesh=_MESH,
    compiler_params=pltpu.CompilerParams(
        needs_layout_passes=False, use_tc_tiling_on_sc=False,
        disable_bounds_checks=True),
    out_type=jax.ShapeDtypeStruct((E,), _f32),
    scratch_types=[
        pltpu.VMEM((EW,), _i32),
        pltpu.VMEM((EW,), _f32),
        pltpu.VMEM((NR, 16), _f32),
        pltpu.VMEM((NR, 16), _f32),
        pltpu.SemaphoreType.DMA,
    ],
)
def _coef(dst_hbm, ex_hbm, den_hbm, cf_hbm, dst_v, cf_v, den_v, dtmp_v, dsem):
    c_ax = lax.axis_index("c")
    s_ax = lax.axis_index("s")
    wid = s_ax * NC + c_ax
    base = wid * EW

    _ins = [(dst_hbm.at[pl.ds(base, EW)], dst_v),
            (ex_hbm.at[pl.ds(base, EW)], cf_v),
            (den_hbm.at[0], den_v), (den_hbm.at[1], dtmp_v)]
    for _s, _d in _ins:
        pltpu.async_copy(_s, _d, dsem)
    for _s, _d in _ins:
        pltpu.make_async_copy(_s, _d, dsem).wait()

    def _sum_den(i, carry):
        den_v[i, :] = den_v[i, :] + dtmp_v[i, :]
        return carry

    lax.fori_loop(0, NR, _sum_den, 0, unroll=8)

    def _cf(i, carry):
        sl = pl.ds(i * 16, 16)
        di = dst_v[sl]
        dn = plsc.load_gather(
            den_v,
            [lax.shift_right_logical(di, 4), lax.bitwise_and(di, 15)],
        )
        cf_v[sl] = cf_v[sl] / (dn + 1e-16)
        return carry

    lax.fori_loop(0, EW // 16, _cf, 0, unroll=5)

    pltpu.sync_copy(cf_v, cf_hbm.at[pl.ds(base, EW)])


_EPI = [(q * SK, min(SK, ROWS_PER_TILE - q * SK))
        for q in range((ROWS_PER_TILE + SK - 1) // SK)]


@functools.partial(
    pl.kernel,
    mesh=_MESH,
    compiler_params=pltpu.CompilerParams(
        needs_layout_passes=False, use_tc_tiling_on_sc=False,
        disable_bounds_checks=True),
    out_type=jax.ShapeDtypeStruct((NC, N, H), _f32),
    scratch_types=[
        pltpu.VMEM((EW,), _i32),
        pltpu.VMEM((NB, SK), _i32),
        pltpu.VMEM((EW,), _f32),
        pltpu.VMEM((SK, H), _f32),
        pltpu.VMEM((SK, H), _f32),
        pltpu.VMEM_SHARED((N, H), _f32),
        pltpu.SemaphoreType.DMA,
        pltpu.SemaphoreType.DMA,
        pltpu.SemaphoreType.DMA,
        pltpu.SemaphoreType.DMA,
    ],
)
def _edge_msg(h_hbm, src_hbm, dst2_hbm, cf_hbm,
              part_hbm,
              src_v, dst2_v, cf_v, rows_a, rows_b, out_sh,
              sem_a, sem_b, sem_sa, sem_sb):
    c_ax = lax.axis_index("c")
    s_ax = lax.axis_index("s")
    wid = s_ax * NC + c_ax
    base = wid * EW
    row0 = s_ax * ROWS_PER_TILE

    _ins = [(src_hbm.at[pl.ds(base, EW)], src_v),
            (dst2_hbm.at[pl.ds(wid * NB, NB)], dst2_v),
            (cf_hbm.at[pl.ds(base, EW)], cf_v)]
    for _s, _d in _ins:
        pltpu.async_copy(_s, _d, sem_a)
    for _s, _d in _ins:
        pltpu.make_async_copy(_s, _d, sem_a).wait()

    def _zrows(i, carry):
        for k in range(H // 16):
            rows_a[i, pl.ds(k * 16, 16)] = jnp.zeros((16,), _f32)
        return carry

    lax.fori_loop(0, SK, _zrows, 0)
    for off, sz in _EPI:
        pltpu.sync_copy(rows_a.at[pl.ds(0, sz)],
                        out_sh.at[pl.ds(row0 + off, sz)])

    plsc.subcore_barrier()

    def _gather(j, buf, sem):
        return pltpu.async_copy(h_hbm.at[src_v.at[pl.ds(j * SK, SK)]],
                                buf, sem)

    def _gwait(j, buf, sem):
        pltpu.make_async_copy(h_hbm.at[src_v.at[pl.ds(j * SK, SK)]],
                              buf, sem).wait()

    def _scale(j, buf):
        def _grp(g, carry2):
            cvec = cf_v[pl.ds(j * SK + g * 16, 16)]
            for lane in range(16):
                cof = cvec[lane]
                r = g * 16 + lane
                for k in range(H // 16):
                    sl = pl.ds(k * 16, 16)
                    buf[r, sl] = buf[r, sl] * cof
            return carry2

        lax.fori_loop(0, SK // 16, _grp, 0)

    def _sstart(j, buf, sem):
        pltpu.async_copy(buf, out_sh.at[dst2_v.at[j]], sem, add=True)

    def _swait(j, buf, sem):
        pltpu.make_async_copy(buf, out_sh.at[dst2_v.at[j]], sem).wait()

    _gather(0, rows_a, sem_a)
    _gather(1, rows_b, sem_b)

    def _pair(j2, carry):
        j0 = 2 * j2
        _gwait(j0, rows_a, sem_a)
        _scale(j0, rows_a)
        _sstart(j0, rows_a, sem_sa)
        _gwait(j0 + 1, rows_b, sem_b)
        _swait(j0, rows_a, sem_sa)
        _gather(j0 + 2, rows_a, sem_a)
        _scale(j0 + 1, rows_b)
        _sstart(j0 + 1, rows_b, sem_sb)
        _swait(j0 + 1, rows_b, sem_sb)
        _gather(j0 + 3, rows_b, sem_b)
        return carry

    lax.fori_loop(0, (NB - 3) // 2, _pair, 0)

    _gwait(NB - 3, rows_a, sem_a)
    _scale(NB - 3, rows_a)
    _sstart(NB - 3, rows_a, sem_sa)
    _gwait(NB - 2, rows_b, sem_b)
    _scale(NB - 2, rows_b)
    _swait(NB - 3, rows_a, sem_sa)
    _gather(NB - 1, rows_a, sem_a)
    _sstart(NB - 2, rows_b, sem_sb)
    _gwait(NB - 1, rows_a, sem_a)
    _scale(NB - 1, rows_a)
    _sstart(NB - 1, rows_a, sem_sa)
    _swait(NB - 2, rows_b, sem_sb)
    _swait(NB - 1, rows_a, sem_sa)

    plsc.subcore_barrier()

    for off, sz in _EPI:
        pltpu.sync_copy(out_sh.at[pl.ds(row0 + off, sz)],
                        rows_a.at[pl.ds(0, sz)])
        pltpu.sync_copy(rows_a.at[pl.ds(0, sz)],
                        part_hbm.at[c_ax, pl.ds(row0 + off, sz)])


def _avec(a_src, a_dst, a_edge):
    av = jnp.zeros((H, 8), _f32)
    return av.at[:, 0].set(a_src).at[:, 1].set(a_dst).at[:, 2].set(a_edge)


def kernel(x, edge_index, edge_weights, W1, a_src1, a_dst1, We1, a_edge1, b1,
           W2, a_src2, a_dst2, We2, a_edge2, b2):
    src = edge_index[0]
    dst = edge_index[1]
    ew = edge_weights[:, 0]
    dst2d = dst.reshape(E // SK, SK)

    h1, al1, ce1 = _dense1(x, W1, _avec(a_src1, a_dst1, a_edge1), We1)
    c16_1 = jnp.broadcast_to(ce1[0, 2], (16,))
    ex1, den1 = _edge_alpha(src, dst, ew,
                            al1[:, 0].reshape(N // 16, 16),
                            al1[:, 1].reshape(N // 16, 16), c16_1)
    cf1 = _coef(dst, ex1, den1)
    part1 = _edge_msg(h1, src, dst2d, cf1)

    h2, al2, ce2 = _dense2(part1[0], part1[1], b1.reshape(1, H), W2,
                           _avec(a_src2, a_dst2, a_edge2), We2)
    c16_2 = jnp.broadcast_to(ce2[0, 2], (16,))
    ex2, den2 = _edge_alpha(src, dst, ew,
                            al2[:, 0].reshape(N // 16, 16),
                            al2[:, 1].reshape(N // 16, 16), c16_2)
    cf2 = _coef(dst, ex2, den2)
    part2 = _edge_msg(h2, src, dst2d, cf2)

    return _final(part2[0], part2[1], b2.reshape(1, H))


# speedup vs baseline: 1.0010x; 1.0010x over previous
import functools

import jax
import jax.numpy as jnp
from jax import lax
from jax.experimental import pallas as pl
from jax.experimental.pallas import tpu as pltpu
from jax.experimental.pallas import tpu_sc as plsc

N = 10000
E = 320000
D = 128
H = 128

NC = 2
NS = 16
NW = NC * NS
EW = E // NW
NR = 640
SK = 80
NB = EW // SK
ROWS_PER_TILE = N // NS

_f32 = jnp.float32
_i32 = jnp.int32


_BLK = 1000
_GRID = N // _BLK


def _dense1_body(x_ref, w_ref, av_ref, we_ref, h_ref, al_ref, ce_ref):
    h = jnp.dot(x_ref[...], w_ref[...], preferred_element_type=_f32)
    h_ref[...] = h
    al_ref[...] = jnp.dot(h, av_ref[...], preferred_element_type=_f32)
    ce_ref[...] = jnp.dot(we_ref[...], av_ref[...], preferred_element_type=_f32)


def _dense1(x, w, av, we):
    return pl.pallas_call(
        _dense1_body,
        grid=(_GRID,),
        in_specs=[
            pl.BlockSpec((_BLK, D), lambda i: (i, 0)),
            pl.BlockSpec((D, H), lambda i: (0, 0)),
            pl.BlockSpec((H, 8), lambda i: (0, 0)),
            pl.BlockSpec((1, H), lambda i: (0, 0)),
        ],
        out_specs=[
            pl.BlockSpec((_BLK, H), lambda i: (i, 0)),
            pl.BlockSpec((_BLK, 8), lambda i: (i, 0)),
            pl.BlockSpec((1, 8), lambda i: (0, 0)),
        ],
        out_shape=[
            jax.ShapeDtypeStruct((N, H), _f32),
            jax.ShapeDtypeStruct((N, 8), _f32),
            jax.ShapeDtypeStruct((1, 8), _f32),
        ],
    )(x, w, av, we)


def _dense2_body(p0_ref, p1_ref, b_ref, w_ref, av_ref, we_ref,
                 h_ref, al_ref, ce_ref):
    xin = jnp.maximum(p0_ref[...] + p1_ref[...] + b_ref[...], 0.0)
    h = jnp.dot(xin, w_ref[...], preferred_element_type=_f32)
    h_ref[...] = h
    al_ref[...] = jnp.dot(h, av_ref[...], preferred_element_type=_f32)
    ce_ref[...] = jnp.dot(we_ref[...], av_ref[...], preferred_element_type=_f32)


def _dense2(p0, p1, b, w, av, we):
    return pl.pallas_call(
        _dense2_body,
        grid=(_GRID,),
        in_specs=[
            pl.BlockSpec((_BLK, H), lambda i: (i, 0)),
            pl.BlockSpec((_BLK, H), lambda i: (i, 0)),
            pl.BlockSpec((1, H), lambda i: (0, 0)),
            pl.BlockSpec((H, H), lambda i: (0, 0)),
            pl.BlockSpec((H, 8), lambda i: (0, 0)),
            pl.BlockSpec((1, H), lambda i: (0, 0)),
        ],
        out_specs=[
            pl.BlockSpec((_BLK, H), lambda i: (i, 0)),
            pl.BlockSpec((_BLK, 8), lambda i: (i, 0)),
            pl.BlockSpec((1, 8), lambda i: (0, 0)),
        ],
        out_shape=[
            jax.ShapeDtypeStruct((N, H), _f32),
            jax.ShapeDtypeStruct((N, 8), _f32),
            jax.ShapeDtypeStruct((1, 8), _f32),
        ],
    )(p0, p1, b, w, av, we)


def _final_body(p0_ref, p1_ref, b_ref, out_ref):
    out_ref[...] = p0_ref[...] + p1_ref[...] + b_ref[...]


def _final(p0, p1, b):
    return pl.pallas_call(
        _final_body,
        grid=(_GRID,),
        in_specs=[
            pl.BlockSpec((_BLK, H), lambda i: (i, 0)),
            pl.BlockSpec((_BLK, H), lambda i: (i, 0)),
            pl.BlockSpec((1, H), lambda i: (0, 0)),
        ],
        out_specs=pl.BlockSpec((_BLK, H), lambda i: (i, 0)),
        out_shape=jax.ShapeDtypeStruct((N, H), _f32),
    )(p0, p1, b)


_MESH = plsc.VectorSubcoreMesh(core_axis_name="c", subcore_axis_name="s")


@functools.partial(
    pl.kernel,
    mesh=_MESH,
    compiler_params=pltpu.CompilerParams(
        needs_layout_passes=False, use_tc_tiling_on_sc=False),
    out_type=[
        jax.ShapeDtypeStruct((E,), _f32),
        jax.ShapeDtypeStruct((NC, NR, 16), _f32),
    ],
    scratch_types=[
        pltpu.VMEM((EW,), _i32),
        pltpu.VMEM((EW,), _i32),
        pltpu.VMEM((EW,), _f32),
        pltpu.VMEM((N // 16, 16), _f32),
        pltpu.VMEM((N // 16, 16), _f32),
        pltpu.VMEM((NR, 16), _f32),
        pltpu.VMEM((16,), _f32),
        pltpu.VMEM((5, 128), _i32),
        pltpu.VMEM_SHARED((NR, 16), _f32),
        pltpu.SemaphoreType.DMA,
    ],
)
def _edge_alpha(src_hbm, dst_hbm, ew_hbm, asrc_hbm, adst_hbm, ce_hbm,
                ex_hbm, den_hbm,
                src_v, dst_v, ew_v, as_v, ad_v, den_v, ce_v, idx_v,
                den_sh, dsem):
    c_ax = lax.axis_index("c")
    s_ax = lax.axis_index("s")
    wid = s_ax * NC + c_ax
    base = wid * EW

    _ins = [(src_hbm.at[pl.ds(base, EW)], src_v),
            (dst_hbm.at[pl.ds(base, EW)], dst_v),
            (ew_hbm.at[pl.ds(base, EW)], ew_v),
            (asrc_hbm, as_v), (adst_hbm, ad_v), (ce_hbm, ce_v)]
    for _s, _d in _ins:
        pltpu.async_copy(_s, _d, dsem)
    for _s, _d in _ins:
        pltpu.make_async_copy(_s, _d, dsem).wait()
    c_e = ce_v[...]

    for j in range(5):
        for t in range(8):
            idx_v[j, pl.ds(t * 16, 16)] = (
                lax.broadcasted_iota(_i32, (16,), 0) + (j * 128 + t * 16)
            )

    def _zero(i, carry):
        den_v[i, :] = jnp.zeros((16,), _f32)
        return carry

    lax.fori_loop(0, NR, _zero, 0, unroll=8)

    @pl.when(s_ax == 0)
    def _init_shared():
        pltpu.sync_copy(den_v, den_sh)

    plsc.subcore_barrier()

    def _edges(i, carry):
        sl = pl.ds(i * 16, 16)
        si = src_v[sl]
        di = dst_v[sl]
        si_r = lax.shift_right_logical(si, 4)
        si_c = lax.bitwise_and(si, 15)
        di_r = lax.shift_right_logical(di, 4)
        di_c = lax.bitwise_and(di, 15)
        a = (plsc.load_gather(as_v, [si_r, si_c])
             + plsc.load_gather(ad_v, [di_r, di_c])
             + c_e * ew_v[sl])
        a = jnp.where(a > 0.0, a, 0.2 * a)
        exv = jnp.exp(a)
        ew_v[sl] = exv
        plsc.addupdate_scatter(den_v, [di_r, di_c], exv)
        return carry

    lax.fori_loop(0, EW // 16, _edges, 0, unroll=5)

    pltpu.sync_copy(ew_v, ex_hbm.at[pl.ds(base, EW)])

    for j in range(5):
        pltpu.sync_copy(den_v.at[pl.ds(j * 128, 128)],
                        den_sh.at[idx_v.at[j]], add=True)

    plsc.subcore_barrier()

    @pl.when(s_ax == 0)
    def _write_out():
        pltpu.sync_copy(den_sh, den_hbm.at[c_ax])


@functools.partial(
    pl.kernel,
    mesh=_MESH,
    compiler_params=pltpu.CompilerParams(
        needs_layout_passes=False, use_tc_tiling_on_sc=False),
    out_type=jax.ShapeDtypeStruct((E,), _f32),
    scratch_types=[
        pltpu.VMEM((EW,), _i32),
        pltpu.VMEM((EW,), _f32),
        pltpu.VMEM((NR, 16), _f32),
        pltpu.VMEM((NR, 16), _f32),
        pltpu.SemaphoreType.DMA,
    ],
)
def _coef(dst_hbm, ex_hbm, den_hbm, cf_hbm, dst_v, cf_v, den_v, dtmp_v, dsem):
    c_ax = lax.axis_index("c")
    s_ax = lax.axis_index("s")
    wid = s_ax * NC + c_ax
    base = wid * EW

    _ins = [(dst_hbm.at[pl.ds(base, EW)], dst_v),
            (ex_hbm.at[pl.ds(base, EW)], cf_v),
            (den_hbm.at[0], den_v), (den_hbm.at[1], dtmp_v)]
    for _s, _d in _ins:
        pltpu.async_copy(_s, _d, dsem)
    for _s, _d in _ins:
        pltpu.make_async_copy(_s, _d, dsem).wait()

    def _sum_den(i, carry):
        den_v[i, :] = den_v[i, :] + dtmp_v[i, :]
        return carry

    lax.fori_loop(0, NR, _sum_den, 0, unroll=8)

    def _cf(i, carry):
        sl = pl.ds(i * 16, 16)
        di = dst_v[sl]
        dn = plsc.load_gather(
            den_v,
            [lax.shift_right_logical(di, 4), lax.bitwise_and(di, 15)],
        )
        cf_v[sl] = cf_v[sl] / (dn + 1e-16)
        return carry

    lax.fori_loop(0, EW // 16, _cf, 0, unroll=5)

    pltpu.sync_copy(cf_v, cf_hbm.at[pl.ds(base, EW)])


_EPI = [(q * SK, min(SK, ROWS_PER_TILE - q * SK))
        for q in range((ROWS_PER_TILE + SK - 1) // SK)]


@functools.partial(
    pl.kernel,
    mesh=_MESH,
    compiler_params=pltpu.CompilerParams(
        needs_layout_passes=False, use_tc_tiling_on_sc=False),
    out_type=jax.ShapeDtypeStruct((NC, N, H), _f32),
    scratch_types=[
        pltpu.VMEM((EW,), _i32),
        pltpu.VMEM((NB, SK), _i32),
        pltpu.VMEM((EW,), _f32),
        pltpu.VMEM((SK, H), _f32),
        pltpu.VMEM((SK, H), _f32),
        pltpu.VMEM_SHARED((N, H), _f32),
        pltpu.SemaphoreType.DMA,
        pltpu.SemaphoreType.DMA,
        pltpu.SemaphoreType.DMA,
        pltpu.SemaphoreType.DMA,
    ],
)
def _edge_msg(h_hbm, src_hbm, dst2_hbm, cf_hbm,
              part_hbm,
              src_v, dst2_v, cf_v, rows_a, rows_b, out_sh,
              sem_a, sem_b, sem_sa, sem_sb):
    c_ax = lax.axis_index("c")
    s_ax = lax.axis_index("s")
    wid = s_ax * NC + c_ax
    base = wid * EW
    row0 = s_ax * ROWS_PER_TILE

    _ins = [(src_hbm.at[pl.ds(base, EW)], src_v),
            (dst2_hbm.at[pl.ds(wid * NB, NB)], dst2_v),
            (cf_hbm.at[pl.ds(base, EW)], cf_v)]
    for _s, _d in _ins:
        pltpu.async_copy(_s, _d, sem_a)
    for _s, _d in _ins:
        pltpu.make_async_copy(_s, _d, sem_a).wait()

    def _zrows(i, carry):
        for k in range(H // 16):
            rows_a[i, pl.ds(k * 16, 16)] = jnp.zeros((16,), _f32)
        return carry

    lax.fori_loop(0, SK, _zrows, 0)
    for off, sz in _EPI:
        pltpu.sync_copy(rows_a.at[pl.ds(0, sz)],
                        out_sh.at[pl.ds(row0 + off, sz)])

    plsc.subcore_barrier()

    def _gather(j, buf, sem):
        return pltpu.async_copy(h_hbm.at[src_v.at[pl.ds(j * SK, SK)]],
                                buf, sem)

    def _gwait(j, buf, sem):
        pltpu.make_async_copy(h_hbm.at[src_v.at[pl.ds(j * SK, SK)]],
                              buf, sem).wait()

    def _scale(j, buf):
        def _grp(g, carry2):
            cvec = cf_v[pl.ds(j * SK + g * 16, 16)]
            for lane in range(16):
                cof = cvec[lane]
                r = g * 16 + lane
                for k in range(H // 16):
                    sl = pl.ds(k * 16, 16)
                    buf[r, sl] = buf[r, sl] * cof
            return carry2

        lax.fori_loop(0, SK // 16, _grp, 0)

    def _sstart(j, buf, sem):
        pltpu.async_copy(buf, out_sh.at[dst2_v.at[j]], sem, add=True)

    def _swait(j, buf, sem):
        pltpu.make_async_copy(buf, out_sh.at[dst2_v.at[j]], sem).wait()

    _gather(0, rows_a, sem_a)
    _gather(1, rows_b, sem_b)

    def _pair(j2, carry):
        j0 = 2 * j2
        _gwait(j0, rows_a, sem_a)
        _scale(j0, rows_a)
        _sstart(j0, rows_a, sem_sa)
        _gwait(j0 + 1, rows_b, sem_b)
        _swait(j0, rows_a, sem_sa)
        _gather(j0 + 2, rows_a, sem_a)
        _scale(j0 + 1, rows_b)
        _sstart(j0 + 1, rows_b, sem_sb)
        _swait(j0 + 1, rows_b, sem_sb)
        _gather(j0 + 3, rows_b, sem_b)
        return carry

    lax.fori_loop(0, (NB - 3) // 2, _pair, 0)

    _gwait(NB - 3, rows_a, sem_a)
    _scale(NB - 3, rows_a)
    _sstart(NB - 3, rows_a, sem_sa)
    _gwait(NB - 2, rows_b, sem_b)
    _scale(NB - 2, rows_b)
    _swait(NB - 3, rows_a, sem_sa)
    _gather(NB - 1, rows_a, sem_a)
    _sstart(NB - 2, rows_b, sem_sb)
    _gwait(NB - 1, rows_a, sem_a)
    _scale(NB - 1, rows_a)
    _sstart(NB - 1, rows_a, sem_sa)
    _swait(NB - 2, rows_b, sem_sb)
    _swait(NB - 1, rows_a, sem_sa)

    plsc.subcore_barrier()

    for off, sz in _EPI:
        pltpu.sync_copy(out_sh.at[pl.ds(row0 + off, sz)],
                        rows_a.at[pl.ds(0, sz)])
        pltpu.sync_copy(rows_a.at[pl.ds(0, sz)],
                        part_hbm.at[c_ax, pl.ds(row0 + off, sz)])


def _avec(a_src, a_dst, a_edge):
    av = jnp.zeros((H, 8), _f32)
    return av.at[:, 0].set(a_src).at[:, 1].set(a_dst).at[:, 2].set(a_edge)


def kernel(x, edge_index, edge_weights, W1, a_src1, a_dst1, We1, a_edge1, b1,
           W2, a_src2, a_dst2, We2, a_edge2, b2):
    src = edge_index[0]
    dst = edge_index[1]
    ew = edge_weights[:, 0]
    dst2d = dst.reshape(E // SK, SK)

    h1, al1, ce1 = _dense1(x, W1, _avec(a_src1, a_dst1, a_edge1), We1)
    c16_1 = jnp.broadcast_to(ce1[0, 2], (16,))
    ex1, den1 = _edge_alpha(src, dst, ew,
                            al1[:, 0].reshape(N // 16, 16),
                            al1[:, 1].reshape(N // 16, 16), c16_1)
    cf1 = _coef(dst, ex1, den1)
    part1 = _edge_msg(h1, src, dst2d, cf1)

    h2, al2, ce2 = _dense2(part1[0], part1[1], b1.reshape(1, H), W2,
                           _avec(a_src2, a_dst2, a_edge2), We2)
    c16_2 = jnp.broadcast_to(ce2[0, 2], (16,))
    ex2, den2 = _edge_alpha(src, dst, ew,
                            al2[:, 0].reshape(N // 16, 16),
                            al2[:, 1].reshape(N // 16, 16), c16_2)
    cf2 = _coef(dst, ex2, den2)
    part2 = _edge_msg(h2, src, dst2d, cf2)

    return _final(part2[0], part2[1], b2.reshape(1, H))
